# Initial kernel scaffold; baseline (speedup 1.0000x reference)
import jax
import jax.numpy as jnp
from jax.experimental import pallas as pl


def kernel(phoneme_input, table, W_ph, b_ph, W_em, b_em, W_vo, b_vo, W_al, b_al):
    raise NotImplementedError("write your pallas kernel here")


# speedup vs baseline: 3.6157x; 3.6157x over previous
import functools

import jax
import jax.numpy as jnp
from jax import lax
from jax.experimental import pallas as pl
from jax.experimental.pallas import tpu as pltpu
from jax.experimental.pallas import tpu_sc as plsc

B = 4096
L = 200
DIM = 64
VOCAB = 100000
ROWS = B * L

NC = 2
NS = 16
NW = NC * NS
ROWS_PER_W = ROWS // NW
PADDIM = 128
CHUNK = 800
N_CHUNKS = ROWS_PER_W // CHUNK

_sc_mesh = plsc.VectorSubcoreMesh(core_axis_name="c", subcore_axis_name="s")


@functools.partial(
    pl.kernel,
    mesh=_sc_mesh,
    out_type=jax.ShapeDtypeStruct((ROWS, PADDIM), jnp.float32),
    scratch_types=[
        pltpu.VMEM((CHUNK,), jnp.int32),
        pltpu.VMEM((CHUNK, PADDIM), jnp.float32),
        pltpu.SemaphoreType.DMA,
    ],
)
def _sc_gather(idx_hbm, table_hbm, out_hbm, idx_v, rows_v, sem):
    wid = lax.axis_index("s") * NC + lax.axis_index("c")
    base = wid * ROWS_PER_W

    def body(i, carry):
        off = base + i * CHUNK
        pltpu.sync_copy(idx_hbm.at[pl.ds(off, CHUNK)], idx_v)
        pltpu.async_copy(table_hbm.at[idx_v], rows_v, sem).wait()
        pltpu.sync_copy(rows_v, out_hbm.at[pl.ds(off, CHUNK)])
        return carry

    lax.fori_loop(0, N_CHUNKS, body, 0)


BLK = 4096


def _dense_body(e_ref, wc_ref, bc_ref, wa_ref, ba_ref, o_ref):
    e = e_ref[:, :DIM]
    h = jnp.tanh(
        jnp.dot(e, wc_ref[...], preferred_element_type=jnp.float32) + bc_ref[...]
    )
    o_ref[...] = (
        e + jnp.dot(h, wa_ref[...], preferred_element_type=jnp.float32) + ba_ref[...]
    )


def _dense(emb, wc, bc, wa, ba):
    grid = (ROWS // BLK,)
    return pl.pallas_call(
        _dense_body,
        grid=grid,
        in_specs=[
            pl.BlockSpec((BLK, PADDIM), lambda i: (i, 0)),
            pl.BlockSpec((DIM, 3 * DIM), lambda i: (0, 0)),
            pl.BlockSpec((1, 3 * DIM), lambda i: (0, 0)),
            pl.BlockSpec((3 * DIM, DIM), lambda i: (0, 0)),
            pl.BlockSpec((1, DIM), lambda i: (0, 0)),
        ],
        out_specs=pl.BlockSpec((BLK, DIM), lambda i: (i, 0)),
        out_shape=jax.ShapeDtypeStruct((ROWS, DIM), jnp.float32),
    )(emb, wc, bc, wa, ba)


def kernel(phoneme_input, table, W_ph, b_ph, W_em, b_em, W_vo, b_vo, W_al, b_al):
    idx = phoneme_input.reshape(-1).astype(jnp.int32)
    table_p = jnp.pad(table, ((0, 0), (0, PADDIM - DIM)))
    emb = _sc_gather(idx, table_p)
    wc = jnp.concatenate([W_ph.T, W_em.T, W_vo.T], axis=1)
    bc = jnp.concatenate([b_ph, b_em, b_vo]).reshape(1, 3 * DIM)
    wa = W_al.T
    ba = b_al.reshape(1, DIM)
    out = _dense(emb, wc, bc, wa, ba)
    return out.reshape(B, L, DIM)
